# baseline (speedup 1.0000x reference)
import functools

import jax
import jax.numpy as jnp
from jax.experimental import pallas as pl
from jax.experimental.pallas import tpu as pltpu

N = 10000
E = 160000
D = 256
H = 8
DH = D // H


def _matmul_bias_kernel(x_ref, w_ref, b_ref, o_ref):
    o_ref[...] = jnp.dot(x_ref[...], w_ref[...],
                         preferred_element_type=jnp.float32) + b_ref[...]


def _matmul_bias(x, w, b, block_rows):
    m, k = x.shape
    _, n = w.shape
    grid = (m // block_rows,)
    return pl.pallas_call(
        _matmul_bias_kernel,
        grid=grid,
        in_specs=[
            pl.BlockSpec((block_rows, k), lambda i: (i, 0)),
            pl.BlockSpec((k, n), lambda i: (0, 0)),
            pl.BlockSpec((n,), lambda i: (0,)),
        ],
        out_specs=pl.BlockSpec((block_rows, n), lambda i: (i, 0)),
        out_shape=jax.ShapeDtypeStruct((m, n), jnp.float32),
    )(x, w, b)


def _ln(x, g, b, eps=1e-5):
    mu = x.mean(axis=-1, keepdims=True)
    var = x.var(axis=-1, keepdims=True)
    return (x - mu) / jnp.sqrt(var + eps) * g + b


def kernel(x, edge_attr, log_deg, Qw, Qb, Kw, Kb, Ew, Eb, Vw, Vb, Aw, VeRow,
           Ohw, Ohb, Oew, Oeb, deg_coef, ln1h_g, ln1h_b, ln1e_g, ln1e_b,
           ln2h_g, ln2h_b, W1, b1, W2, b2, edge_index):
    ei0 = edge_index[0]
    ei1 = edge_index[1]
    Qh = _matmul_bias(x, Qw, Qb, 1000).reshape(N, H, DH)
    Kh = _matmul_bias(x, Kw, Kb, 1000).reshape(N, H, DH)
    Vh = _matmul_bias(x, Vw, Vb, 1000).reshape(N, H, DH)
    Ee = _matmul_bias(edge_attr, Ew, Eb, 640).reshape(E, H, 2 * DH)
    E_w = Ee[:, :, :DH]
    E_b = Ee[:, :, DH:]
    score = Kh[ei0] + Qh[ei1]
    score = score * E_w
    score = jnp.sqrt(jax.nn.relu(score) + 1e-8) - jnp.sqrt(jax.nn.relu(-score) + 1e-8)
    score = score + E_b
    score = jax.nn.relu(score)
    e_t = score
    wE = score.reshape(E, D)
    sc = jnp.einsum('ehd,dhc->ehc', score, Aw)
    sc = jnp.clip(sc, -5.0, 5.0)
    sexp = jnp.exp(sc)
    ssum = jax.ops.segment_sum(sexp, ei1, num_segments=N)
    attn = sexp / (ssum[ei1] + 1e-16)
    msg = Vh[ei0] * attn
    wV = jax.ops.segment_sum(msg, ei1, num_segments=N)
    rowV = jax.ops.segment_sum(e_t * attn, ei1, num_segments=N)
    rowV = jnp.einsum('nhd,dhc->nhc', rowV, VeRow)
    wV = wV + rowV
    h = wV.reshape(N, D)
    h = jnp.stack([h, h * log_deg.reshape(N, 1)], axis=-1)
    h = (h * deg_coef).sum(axis=-1)
    h = _matmul_bias(h, Ohw, Ohb, 1000)
    e = _matmul_bias(wE, Oew, Oeb, 640)
    h = x + h
    e = e + edge_attr
    h = _ln(h, ln1h_g, ln1h_b)
    e = _ln(e, ln1e_g, ln1e_b)
    h2 = _matmul_bias(jax.nn.relu(_matmul_bias(h, W1, b1, 1000)), W2, b2, 1000)
    h = _ln(h + h2, ln2h_g, ln2h_b)
    return (h, e)


# speedup vs baseline: 13.2077x; 13.2077x over previous
import functools

import jax
import jax.numpy as jnp
import numpy as np
from jax import lax
from jax.experimental import pallas as pl
from jax.experimental.pallas import tpu as pltpu
from jax.experimental.pallas import tpu_sc as plsc

N = 10000
E = 160000
D = 256
H = 8
DH = D // H

CHUNK = 128
NCH = E // CHUNK
NW = 32
TPC = 632
N_PAD = 16 * TPC


def _matmul_bias_kernel(x_ref, w_ref, b_ref, o_ref):
    o_ref[...] = jnp.dot(x_ref[...], w_ref[...],
                         preferred_element_type=jnp.float32) + b_ref[...]


def _matmul_bias(x, w, b, block_rows):
    m, k = x.shape
    _, n = w.shape
    return pl.pallas_call(
        _matmul_bias_kernel,
        grid=(m // block_rows,),
        in_specs=[
            pl.BlockSpec((block_rows, k), lambda i: (i, 0)),
            pl.BlockSpec((k, n), lambda i: (0, 0)),
            pl.BlockSpec((1, n), lambda i: (0, 0)),
        ],
        out_specs=pl.BlockSpec((block_rows, n), lambda i: (i, 0)),
        out_shape=jax.ShapeDtypeStruct((m, n), jnp.float32),
    )(x, w, b.reshape(1, n))


def _sc_gather(kvh, qh, ei0_2d, ei1_2d):
    mesh = plsc.VectorSubcoreMesh(core_axis_name="c", subcore_axis_name="s")

    @functools.partial(
        pl.kernel,
        out_type=(jax.ShapeDtypeStruct((E, 2 * D), jnp.float32),
                  jax.ShapeDtypeStruct((E, D), jnp.float32)),
        mesh=mesh,
        scratch_types=[
            pltpu.VMEM((CHUNK,), jnp.int32),
            pltpu.VMEM((CHUNK,), jnp.int32),
            pltpu.VMEM((CHUNK, 2 * D), jnp.float32),
            pltpu.VMEM((CHUNK, D), jnp.float32),
            pltpu.SemaphoreType.DMA,
            pltpu.SemaphoreType.DMA,
        ],
    )
    def g1(kvh_h, qh_h, ei0_h, ei1_h, kvg_out, qg_out,
           idx0_v, idx1_v, kv_v, q_v, sem0, sem1):
        c = lax.axis_index("c")
        s = lax.axis_index("s")
        wid = s * 2 + c
        nch = (NCH - wid + NW - 1) // NW

        def body(i, carry):
            j = wid + i * NW
            pltpu.sync_copy(ei0_h.at[j], idx0_v)
            pltpu.sync_copy(ei1_h.at[j], idx1_v)
            cp0 = pltpu.async_copy(kvh_h.at[idx0_v], kv_v, sem0)
            cp1 = pltpu.async_copy(qh_h.at[idx1_v], q_v, sem1)
            cp0.wait()
            cp1.wait()
            pltpu.sync_copy(kv_v, kvg_out.at[pl.ds(j * CHUNK, CHUNK)])
            pltpu.sync_copy(q_v, qg_out.at[pl.ds(j * CHUNK, CHUNK)])
            return carry

        lax.fori_loop(0, nch, body, 0)

    return g1(kvh, qh, ei0_2d, ei1_2d)


SCHUNK = 64
NSCH = E // SCHUNK


_SC_LEVEL = 0


def _sc_scatter(ei1_2d, m0, m1, m2, m3, sexp16):
    mesh = plsc.VectorSubcoreMesh(core_axis_name="c", subcore_axis_name="s")
    pf32 = jnp.float32
    ei1_s = ei1_2d.reshape(NSCH, SCHUNK)
    zeros128 = jnp.zeros((N_PAD, 128), pf32)
    zeros16 = jnp.zeros((N_PAD, 16), pf32)

    @functools.partial(
        pl.kernel,
        out_type=(jax.ShapeDtypeStruct((2, N_PAD, 128), pf32),
                  jax.ShapeDtypeStruct((2, N_PAD, 128), pf32),
                  jax.ShapeDtypeStruct((2, N_PAD, 128), pf32),
                  jax.ShapeDtypeStruct((2, N_PAD, 128), pf32),
                  jax.ShapeDtypeStruct((2, N_PAD, 16), pf32)),
        mesh=mesh,
        scratch_types=[
            pltpu.VMEM((1, SCHUNK), jnp.int32),
            pltpu.VMEM((SCHUNK, 128), pf32),
            pltpu.VMEM((SCHUNK, 16), pf32),
            pltpu.VMEM_SHARED((N_PAD, 128), pf32),
            pltpu.VMEM_SHARED((N_PAD, 16), pf32),
        ],
    )
    def sck(ei1_h, m0_h, m1_h, m2_h, m3_h, sx_h, z128_h, z16_h,
            p0_out, p1_out, p2_out, p3_out, ps_out,
            idxs_v, data_v, sx_v, acc, accs):
        c = lax.axis_index("c")
        s = lax.axis_index("s")
        wid = s * 2 + c
        nch = (NSCH - wid + NW - 1) // NW

        def run_group(src_h, out_h, accum, stage_v, zero_h, ncols):
            if _SC_LEVEL >= 2:
                @pl.when(s == 0)
                def _():
                    pltpu.sync_copy(zero_h, accum)

            if _SC_LEVEL >= 3:
                plsc.subcore_barrier()

            if _SC_LEVEL >= 5:
                def body(i, carry):
                    j = wid + i * NW
                    pltpu.sync_copy(ei1_h.at[j], idxs_v.at[0])
                    pltpu.sync_copy(src_h.at[pl.ds(j * SCHUNK, SCHUNK)],
                                    stage_v)
                    pltpu.sync_copy(stage_v, accum.at[idxs_v.at[0]], add=True)
                    return carry
                lax.fori_loop(0, nch, body, 0)
            if _SC_LEVEL >= 3:
                plsc.subcore_barrier()

            if _SC_LEVEL >= 4:
                @pl.when(s == 0)
                def _():
                    pltpu.sync_copy(accum, out_h.at[c])

        run_group(m0_h, p0_out, acc, data_v, z128_h, 128)
        run_group(m1_h, p1_out, acc, data_v, z128_h, 128)
        run_group(m2_h, p2_out, acc, data_v, z128_h, 128)
        run_group(m3_h, p3_out, acc, data_v, z128_h, 128)
        run_group(sx_h, ps_out, accs, sx_v, z16_h, 16)

    return sck(ei1_s, m0, m1, m2, m3, sexp16, zeros128, zeros16)


def _edge_stage_kernel(kvg_ref, qg_ref, ew_ref, ea_ref,
                       awbd_ref, expand_ref, oew_ref, oeb_ref,
                       g_ref, b_ref,
                       m0_ref, m1_ref, m2_ref, m3_ref, sx_ref, e_ref):
    kg = kvg_ref[:, :D]
    vg = kvg_ref[:, D:]
    ewf = ew_ref[:, :D]
    ebf = ew_ref[:, D:]
    s = (kg + qg_ref[...]) * ewf
    s = jnp.sqrt(jax.nn.relu(s) + 1e-8) - jnp.sqrt(jax.nn.relu(-s) + 1e-8)
    e_t = jax.nn.relu(s + ebf)
    sc = jnp.dot(e_t, awbd_ref[...], preferred_element_type=jnp.float32)
    sc = jnp.clip(sc, -5.0, 5.0)
    sexp = jnp.exp(sc)
    sx_ref[...] = jnp.concatenate(
        [sexp, jnp.zeros_like(sexp)], axis=1)
    sexpex = jnp.dot(sexp, expand_ref[...], preferred_element_type=jnp.float32)
    umsg = vg * sexpex
    uwrow = e_t * sexpex
    m0_ref[...] = umsg[:, :128]
    m1_ref[...] = umsg[:, 128:]
    m2_ref[...] = uwrow[:, :128]
    m3_ref[...] = uwrow[:, 128:]
    ev = jnp.dot(e_t, oew_ref[...], preferred_element_type=jnp.float32)
    ev = ev + oeb_ref[...] + ea_ref[...]
    mu = ev.mean(axis=-1, keepdims=True)
    var = ev.var(axis=-1, keepdims=True)
    e_ref[...] = (ev - mu) / jnp.sqrt(var + 1e-5) * g_ref[...] + b_ref[...]


def _edge_stage(kvg, qg, ewcat, edge_attr, awbd, expand, oew, oeb, g, b):
    BE = 640
    grid = (E // BE,)
    row = lambda i: (i, 0)
    fixed = lambda i: (0, 0)
    return pl.pallas_call(
        _edge_stage_kernel,
        grid=grid,
        in_specs=[
            pl.BlockSpec((BE, 2 * D), row),
            pl.BlockSpec((BE, D), row),
            pl.BlockSpec((BE, 2 * D), row),
            pl.BlockSpec((BE, D), row),
            pl.BlockSpec((D, H), fixed),
            pl.BlockSpec((H, D), fixed),
            pl.BlockSpec((D, D), fixed),
            pl.BlockSpec((1, D), fixed),
            pl.BlockSpec((1, D), fixed),
            pl.BlockSpec((1, D), fixed),
        ],
        out_specs=[
            pl.BlockSpec((BE, 128), row),
            pl.BlockSpec((BE, 128), row),
            pl.BlockSpec((BE, 128), row),
            pl.BlockSpec((BE, 128), row),
            pl.BlockSpec((BE, 16), row),
            pl.BlockSpec((BE, D), row),
        ],
        out_shape=[
            jax.ShapeDtypeStruct((E, 128), jnp.float32),
            jax.ShapeDtypeStruct((E, 128), jnp.float32),
            jax.ShapeDtypeStruct((E, 128), jnp.float32),
            jax.ShapeDtypeStruct((E, 128), jnp.float32),
            jax.ShapeDtypeStruct((E, 16), jnp.float32),
            jax.ShapeDtypeStruct((E, D), jnp.float32),
        ],
    )(kvg, qg, ewcat, edge_attr, awbd, expand, oew,
      oeb.reshape(1, D), g.reshape(1, D), b.reshape(1, D))


def _node_stage_kernel(p0_ref, p1_ref, p2_ref, p3_ref, ps_ref,
                       x_ref, ld_ref, dc0_ref, dc1_ref,
                       expand_ref, vebd_ref, ohw_ref, ohb_ref,
                       ln1g_ref, ln1b_ref, ln2g_ref, ln2b_ref,
                       w1_ref, b1_ref, w2_ref, b2_ref, h_ref):
    ssum = ps_ref[0, :, :H] + ps_ref[1, :, :H]
    rsum = 1.0 / (ssum + 1e-16)
    rex = jnp.dot(rsum, expand_ref[...], preferred_element_type=jnp.float32)
    msum = jnp.concatenate([p0_ref[0] + p0_ref[1], p1_ref[0] + p1_ref[1]],
                           axis=1)
    wrsum = jnp.concatenate([p2_ref[0] + p2_ref[1], p3_ref[0] + p3_ref[1]],
                            axis=1)
    wv = msum * rex
    wrow = wrsum * rex
    wv = wv + jnp.dot(wrow, vebd_ref[...], preferred_element_type=jnp.float32)
    ld = ld_ref[...]
    hh = wv * (dc0_ref[...] + ld * dc1_ref[...])
    hh = jnp.dot(hh, ohw_ref[...], preferred_element_type=jnp.float32)
    hh = hh + ohb_ref[...] + x_ref[...]
    mu = hh.mean(axis=-1, keepdims=True)
    var = hh.var(axis=-1, keepdims=True)
    u = (hh - mu) / jnp.sqrt(var + 1e-5) * ln1g_ref[...] + ln1b_ref[...]
    t = jax.nn.relu(jnp.dot(u, w1_ref[...],
                            preferred_element_type=jnp.float32) + b1_ref[...])
    h2 = jnp.dot(t, w2_ref[...], preferred_element_type=jnp.float32) + b2_ref[...]
    v = u + h2
    mu = v.mean(axis=-1, keepdims=True)
    var = v.var(axis=-1, keepdims=True)
    h_ref[...] = (v - mu) / jnp.sqrt(var + 1e-5) * ln2g_ref[...] + ln2b_ref[...]


def _node_stage(p0, p1, p2, p3, ps, x, log_deg, dc0, dc1, expand, vebd,
                ohw, ohb, ln1g, ln1b, ln2g, ln2b, w1, b1, w2, b2):
    BN = 1000
    grid = (N // BN,)
    prow = lambda i: (0, i, 0)
    row = lambda i: (i, 0)
    fixed = lambda i: (0, 0)
    r1 = lambda v: v.reshape(1, -1)
    return pl.pallas_call(
        _node_stage_kernel,
        grid=grid,
        in_specs=[
            pl.BlockSpec((2, BN, 128), prow),
            pl.BlockSpec((2, BN, 128), prow),
            pl.BlockSpec((2, BN, 128), prow),
            pl.BlockSpec((2, BN, 128), prow),
            pl.BlockSpec((2, BN, 16), prow),
            pl.BlockSpec((BN, D), row),
            pl.BlockSpec((BN, 1), row),
            pl.BlockSpec((1, D), fixed),
            pl.BlockSpec((1, D), fixed),
            pl.BlockSpec((H, D), fixed),
            pl.BlockSpec((D, D), fixed),
            pl.BlockSpec((D, D), fixed),
            pl.BlockSpec((1, D), fixed),
            pl.BlockSpec((1, D), fixed),
            pl.BlockSpec((1, D), fixed),
            pl.BlockSpec((1, D), fixed),
            pl.BlockSpec((1, D), fixed),
            pl.BlockSpec((D, 2 * D), fixed),
            pl.BlockSpec((1, 2 * D), fixed),
            pl.BlockSpec((2 * D, D), fixed),
            pl.BlockSpec((1, D), fixed),
        ],
        out_specs=pl.BlockSpec((BN, D), row),
        out_shape=jax.ShapeDtypeStruct((N, D), jnp.float32),
    )(p0, p1, p2, p3, ps, x, log_deg.reshape(N, 1), r1(dc0), r1(dc1),
      expand, vebd, ohw, r1(ohb), r1(ln1g), r1(ln1b), r1(ln2g), r1(ln2b),
      w1, r1(b1), w2, r1(b2))


_PERM = np.array([h * 64 + d for h in range(H) for d in range(DH)]
                 + [h * 64 + DH + d for h in range(H) for d in range(DH)],
                 dtype=np.int32)


def kernel(x, edge_attr, log_deg, Qw, Qb, Kw, Kb, Ew, Eb, Vw, Vb, Aw, VeRow,
           Ohw, Ohb, Oew, Oeb, deg_coef, ln1h_g, ln1h_b, ln1e_g, ln1e_b,
           ln2h_g, ln2h_b, W1, b1, W2, b2, edge_index):
    ei0_2d = edge_index[0].reshape(NCH, CHUNK)
    ei1_2d = edge_index[1].reshape(NCH, CHUNK)
    kvw = jnp.concatenate([Kw, Vw], axis=1)
    kvb = jnp.concatenate([Kb, Vb], axis=0)
    ew_p = Ew[:, _PERM]
    eb_p = Eb[_PERM]
    a_hd = Aw[:, :, 0].T
    awbd = (a_hd[:, :, None] * jnp.eye(H, dtype=jnp.float32)[:, None, :])
    awbd = awbd.reshape(D, H)
    expand = jnp.repeat(jnp.eye(H, dtype=jnp.float32), DH, axis=1)
    vebd = (jnp.eye(H, dtype=jnp.float32)[:, None, :, None]
            * jnp.transpose(VeRow, (1, 0, 2))[:, :, None, :]).reshape(D, D)
    dc0 = deg_coef[0, :, 0]
    dc1 = deg_coef[0, :, 1]

    kvh = _matmul_bias(x, kvw, kvb, 1000)
    qh = _matmul_bias(x, Qw, Qb, 1000)
    ewcat = _matmul_bias(edge_attr, ew_p, eb_p, 640)

    kvg, qg = _sc_gather(kvh, qh, ei0_2d, ei1_2d)

    m0, m1, m2, m3, sexp16, e = _edge_stage(
        kvg, qg, ewcat, edge_attr, awbd, expand, Oew, Oeb, ln1e_g, ln1e_b)

    if _SC_LEVEL >= 1:
        p0, p1, p2, p3, ps = _sc_scatter(ei1_2d, m0, m1, m2, m3, sexp16)
        p0, p1, p2, p3, ps = (v[:, :N] for v in (p0, p1, p2, p3, ps))
    else:
        zz5 = jnp.zeros((2, N, 128), jnp.float32)
        p0 = p1 = p2 = p3 = zz5
        ps = jnp.zeros((2, N, 16), jnp.float32)
    if _SC_LEVEL < 5:
        ei1 = edge_index[1]
        seg = lambda v: jax.ops.segment_sum(v, ei1, num_segments=N)
        zz = jnp.zeros_like
        p0 = p0 + jnp.stack([seg(m0), zz(p0[1])])
        p1 = p1 + jnp.stack([seg(m1), zz(p1[1])])
        p2 = p2 + jnp.stack([seg(m2), zz(p2[1])])
        p3 = p3 + jnp.stack([seg(m3), zz(p3[1])])
        ps = ps + jnp.stack([seg(sexp16), zz(ps[1])])

    h = _node_stage(p0, p1, p2, p3, ps, x, log_deg, dc0, dc1, expand, vebd,
                    Ohw, Ohb, ln1h_g, ln1h_b, ln2h_g, ln2h_b, W1, b1, W2, b2)
    return (h, e)


# speedup vs baseline: 13.2755x; 1.0051x over previous
import functools

import jax
import jax.numpy as jnp
import numpy as np
from jax import lax
from jax.experimental import pallas as pl
from jax.experimental.pallas import tpu as pltpu
from jax.experimental.pallas import tpu_sc as plsc

N = 10000
E = 160000
D = 256
H = 8
DH = D // H

CHUNK = 128
NCH = E // CHUNK
NW = 32


def _matmul_bias_kernel(x_ref, w_ref, b_ref, o_ref):
    o_ref[...] = jnp.dot(x_ref[...], w_ref[...],
                         preferred_element_type=jnp.float32) + b_ref[...]


def _matmul_bias(x, w, b, block_rows):
    m, k = x.shape
    _, n = w.shape
    return pl.pallas_call(
        _matmul_bias_kernel,
        grid=(m // block_rows,),
        in_specs=[
            pl.BlockSpec((block_rows, k), lambda i: (i, 0)),
            pl.BlockSpec((k, n), lambda i: (0, 0)),
            pl.BlockSpec((1, n), lambda i: (0, 0)),
        ],
        out_specs=pl.BlockSpec((block_rows, n), lambda i: (i, 0)),
        out_shape=jax.ShapeDtypeStruct((m, n), jnp.float32),
    )(x, w, b.reshape(1, n))


def _sc_gather(kvh, qh, ei0_2d, ei1_2d):
    mesh = plsc.VectorSubcoreMesh(core_axis_name="c", subcore_axis_name="s")

    @functools.partial(
        pl.kernel,
        out_type=(jax.ShapeDtypeStruct((E, 2 * D), jnp.float32),
                  jax.ShapeDtypeStruct((E, D), jnp.float32)),
        mesh=mesh,
        scratch_types=[
            pltpu.VMEM((CHUNK,), jnp.int32),
            pltpu.VMEM((CHUNK,), jnp.int32),
            pltpu.VMEM((CHUNK, 2 * D), jnp.float32),
            pltpu.VMEM((CHUNK, D), jnp.float32),
            pltpu.SemaphoreType.DMA,
            pltpu.SemaphoreType.DMA,
        ],
    )
    def g1(kvh_h, qh_h, ei0_h, ei1_h, kvg_out, qg_out,
           idx0_v, idx1_v, kv_v, q_v, sem0, sem1):
        c = lax.axis_index("c")
        s = lax.axis_index("s")
        wid = s * 2 + c
        nch = (NCH - wid + NW - 1) // NW

        def body(i, carry):
            j = wid + i * NW
            pltpu.sync_copy(ei0_h.at[j], idx0_v)
            pltpu.sync_copy(ei1_h.at[j], idx1_v)
            cp0 = pltpu.async_copy(kvh_h.at[idx0_v], kv_v, sem0)
            cp1 = pltpu.async_copy(qh_h.at[idx1_v], q_v, sem1)
            cp0.wait()
            cp1.wait()
            pltpu.sync_copy(kv_v, kvg_out.at[pl.ds(j * CHUNK, CHUNK)])
            pltpu.sync_copy(q_v, qg_out.at[pl.ds(j * CHUNK, CHUNK)])
            return carry

        lax.fori_loop(0, nch, body, 0)

    return g1(kvh, qh, ei0_2d, ei1_2d)


def _edge_stage_kernel(kvg_ref, qg_ref, ew_ref, ea_ref,
                       awbd_ref, expand_ref, oew_ref, oeb_ref,
                       g_ref, b_ref,
                       m0_ref, m1_ref, m2_ref, m3_ref, sx_ref, e_ref):
    kg = kvg_ref[:, :D]
    vg = kvg_ref[:, D:]
    ewf = ew_ref[:, :D]
    ebf = ew_ref[:, D:]
    s = (kg + qg_ref[...]) * ewf
    s = jnp.sqrt(jax.nn.relu(s) + 1e-8) - jnp.sqrt(jax.nn.relu(-s) + 1e-8)
    e_t = jax.nn.relu(s + ebf)
    sc = jnp.dot(e_t, awbd_ref[...], preferred_element_type=jnp.float32)
    sc = jnp.clip(sc, -5.0, 5.0)
    sexp = jnp.exp(sc)
    sx_ref[...] = jnp.concatenate([sexp, jnp.zeros_like(sexp)], axis=1)
    sexpex = jnp.dot(sexp, expand_ref[...], preferred_element_type=jnp.float32)
    umsg = vg * sexpex
    uwrow = e_t * sexpex
    m0_ref[...] = umsg[:, :128]
    m1_ref[...] = umsg[:, 128:]
    m2_ref[...] = uwrow[:, :128]
    m3_ref[...] = uwrow[:, 128:]
    ev = jnp.dot(e_t, oew_ref[...], preferred_element_type=jnp.float32)
    ev = ev + oeb_ref[...] + ea_ref[...]
    mu = ev.mean(axis=-1, keepdims=True)
    var = ev.var(axis=-1, keepdims=True)
    e_ref[...] = (ev - mu) / jnp.sqrt(var + 1e-5) * g_ref[...] + b_ref[...]


def _edge_stage(kvg, qg, ewcat, edge_attr, awbd, expand, oew, oeb, g, b):
    BE = 640
    row = lambda i: (i, 0)
    fixed = lambda i: (0, 0)
    return pl.pallas_call(
        _edge_stage_kernel,
        grid=(E // BE,),
        in_specs=[
            pl.BlockSpec((BE, 2 * D), row),
            pl.BlockSpec((BE, D), row),
            pl.BlockSpec((BE, 2 * D), row),
            pl.BlockSpec((BE, D), row),
            pl.BlockSpec((D, H), fixed),
            pl.BlockSpec((H, D), fixed),
            pl.BlockSpec((D, D), fixed),
            pl.BlockSpec((1, D), fixed),
            pl.BlockSpec((1, D), fixed),
            pl.BlockSpec((1, D), fixed),
        ],
        out_specs=[
            pl.BlockSpec((BE, 128), row),
            pl.BlockSpec((BE, 128), row),
            pl.BlockSpec((BE, 128), row),
            pl.BlockSpec((BE, 128), row),
            pl.BlockSpec((BE, 16), row),
            pl.BlockSpec((BE, D), row),
        ],
        out_shape=[
            jax.ShapeDtypeStruct((E, 128), jnp.float32),
            jax.ShapeDtypeStruct((E, 128), jnp.float32),
            jax.ShapeDtypeStruct((E, 128), jnp.float32),
            jax.ShapeDtypeStruct((E, 128), jnp.float32),
            jax.ShapeDtypeStruct((E, 16), jnp.float32),
            jax.ShapeDtypeStruct((E, D), jnp.float32),
        ],
    )(kvg, qg, ewcat, edge_attr, awbd, expand, oew,
      oeb.reshape(1, D), g.reshape(1, D), b.reshape(1, D))


def _node_stage_kernel(p0_ref, p1_ref, p2_ref, p3_ref, ps_ref,
                       x_ref, ld_ref, dc0_ref, dc1_ref,
                       expand_ref, vebd_ref, ohw_ref, ohb_ref,
                       ln1g_ref, ln1b_ref, ln2g_ref, ln2b_ref,
                       w1_ref, b1_ref, w2_ref, b2_ref, h_ref):
    ssum = ps_ref[:, :H]
    rsum = 1.0 / (ssum + 1e-16)
    rex = jnp.dot(rsum, expand_ref[...], preferred_element_type=jnp.float32)
    msum = jnp.concatenate([p0_ref[...], p1_ref[...]], axis=1)
    wrsum = jnp.concatenate([p2_ref[...], p3_ref[...]], axis=1)
    wv = msum * rex
    wrow = wrsum * rex
    wv = wv + jnp.dot(wrow, vebd_ref[...], preferred_element_type=jnp.float32)
    ld = ld_ref[...]
    hh = wv * (dc0_ref[...] + ld * dc1_ref[...])
    hh = jnp.dot(hh, ohw_ref[...], preferred_element_type=jnp.float32)
    hh = hh + ohb_ref[...] + x_ref[...]
    mu = hh.mean(axis=-1, keepdims=True)
    var = hh.var(axis=-1, keepdims=True)
    u = (hh - mu) / jnp.sqrt(var + 1e-5) * ln1g_ref[...] + ln1b_ref[...]
    t = jax.nn.relu(jnp.dot(u, w1_ref[...],
                            preferred_element_type=jnp.float32) + b1_ref[...])
    h2 = jnp.dot(t, w2_ref[...],
                 preferred_element_type=jnp.float32) + b2_ref[...]
    v = u + h2
    mu = v.mean(axis=-1, keepdims=True)
    var = v.var(axis=-1, keepdims=True)
    h_ref[...] = (v - mu) / jnp.sqrt(var + 1e-5) * ln2g_ref[...] + ln2b_ref[...]


def _node_stage(p0, p1, p2, p3, ps, x, log_deg, dc0, dc1, expand, vebd,
                ohw, ohb, ln1g, ln1b, ln2g, ln2b, w1, b1, w2, b2):
    BN = 1000
    row = lambda i: (i, 0)
    fixed = lambda i: (0, 0)
    r1 = lambda v: v.reshape(1, -1)
    return pl.pallas_call(
        _node_stage_kernel,
        grid=(N // BN,),
        in_specs=[
            pl.BlockSpec((BN, 128), row),
            pl.BlockSpec((BN, 128), row),
            pl.BlockSpec((BN, 128), row),
            pl.BlockSpec((BN, 128), row),
            pl.BlockSpec((BN, 16), row),
            pl.BlockSpec((BN, D), row),
            pl.BlockSpec((BN, 1), row),
            pl.BlockSpec((1, D), fixed),
            pl.BlockSpec((1, D), fixed),
            pl.BlockSpec((H, D), fixed),
            pl.BlockSpec((D, D), fixed),
            pl.BlockSpec((D, D), fixed),
            pl.BlockSpec((1, D), fixed),
            pl.BlockSpec((1, D), fixed),
            pl.BlockSpec((1, D), fixed),
            pl.BlockSpec((1, D), fixed),
            pl.BlockSpec((1, D), fixed),
            pl.BlockSpec((D, 2 * D), fixed),
            pl.BlockSpec((1, 2 * D), fixed),
            pl.BlockSpec((2 * D, D), fixed),
            pl.BlockSpec((1, D), fixed),
        ],
        out_specs=pl.BlockSpec((BN, D), row),
        out_shape=jax.ShapeDtypeStruct((N, D), jnp.float32),
    )(p0, p1, p2, p3, ps, x, log_deg.reshape(N, 1), r1(dc0), r1(dc1),
      expand, vebd, ohw, r1(ohb), r1(ln1g), r1(ln1b), r1(ln2g), r1(ln2b),
      w1, r1(b1), w2, r1(b2))


_PERM = np.array([h * 64 + d for h in range(H) for d in range(DH)]
                 + [h * 64 + DH + d for h in range(H) for d in range(DH)],
                 dtype=np.int32)


def kernel(x, edge_attr, log_deg, Qw, Qb, Kw, Kb, Ew, Eb, Vw, Vb, Aw, VeRow,
           Ohw, Ohb, Oew, Oeb, deg_coef, ln1h_g, ln1h_b, ln1e_g, ln1e_b,
           ln2h_g, ln2h_b, W1, b1, W2, b2, edge_index):
    ei0_2d = edge_index[0].reshape(NCH, CHUNK)
    ei1_2d = edge_index[1].reshape(NCH, CHUNK)
    kvw = jnp.concatenate([Kw, Vw], axis=1)
    kvb = jnp.concatenate([Kb, Vb], axis=0)
    ew_p = Ew[:, _PERM]
    eb_p = Eb[_PERM]
    a_hd = Aw[:, :, 0].T
    awbd = (a_hd[:, :, None] * jnp.eye(H, dtype=jnp.float32)[:, None, :])
    awbd = awbd.reshape(D, H)
    expand = jnp.repeat(jnp.eye(H, dtype=jnp.float32), DH, axis=1)
    vebd = (jnp.eye(H, dtype=jnp.float32)[:, None, :, None]
            * jnp.transpose(VeRow, (1, 0, 2))[:, :, None, :]).reshape(D, D)
    dc0 = deg_coef[0, :, 0]
    dc1 = deg_coef[0, :, 1]

    kvh = _matmul_bias(x, kvw, kvb, 1000)
    qh = _matmul_bias(x, Qw, Qb, 1000)
    ewcat = _matmul_bias(edge_attr, ew_p, eb_p, 640)

    kvg, qg = _sc_gather(kvh, qh, ei0_2d, ei1_2d)

    m0, m1, m2, m3, sexp16, e = _edge_stage(
        kvg, qg, ewcat, edge_attr, awbd, expand, Oew, Oeb, ln1e_g, ln1e_b)

    ei1 = edge_index[1]
    seg = lambda v: jax.ops.segment_sum(v, ei1, num_segments=N)
    p0, p1, p2, p3, ps = seg(m0), seg(m1), seg(m2), seg(m3), seg(sexp16)

    h = _node_stage(p0, p1, p2, p3, ps, x, log_deg, dc0, dc1, expand, vebd,
                    Ohw, Ohb, ln1h_g, ln1h_b, ln2h_g, ln2h_b, W1, b1, W2, b2)
    return (h, e)
